# Initial kernel scaffold; baseline (speedup 1.0000x reference)
import jax
import jax.numpy as jnp
from jax.experimental import pallas as pl


def kernel(x, W_gate, b_gate):
    raise NotImplementedError("write your pallas kernel here")


# speedup vs baseline: 1.4938x; 1.4938x over previous
import functools

import jax
import jax.numpy as jnp
from jax.experimental import pallas as pl
from jax.experimental.pallas import tpu as pltpu

TOPK = 8
BLK = 1024


def _router_body(x_ref, w_ref, b_ref, idx_ref, val_ref, sum_ref, *, n_experts):
    logits = jnp.dot(x_ref[...], w_ref[...],
                     preferred_element_type=jnp.float32) + b_ref[...]
    m = jnp.max(logits, axis=-1, keepdims=True)
    e = jnp.exp(logits - m)
    score = e / jnp.sum(e, axis=-1, keepdims=True)

    iota = jax.lax.broadcasted_iota(jnp.int32, score.shape, 1)
    work = score
    vals = []
    idxs = []
    for _ in range(TOPK):
        mx = jnp.max(work, axis=-1)
        amx = jnp.argmax(work, axis=-1)
        vals.append(mx)
        idxs.append(amx)
        work = jnp.where(iota == amx[:, None], -1.0, work)
    val = jnp.stack(vals, axis=-1)
    idx = jnp.stack(idxs, axis=-1).astype(jnp.int32)

    idx_ref[...] = idx
    val_ref[...] = val

    @pl.when(pl.program_id(0) == 0)
    def _init():
        sum_ref[0] = 0.0

    sum_ref[0] += jnp.sum(val)


def _norm_body(val_ref, sum_ref, out_ref):
    out_ref[...] = val_ref[...] * (1.0 / sum_ref[0])


@jax.jit
def kernel(x, W_gate, b_gate):
    n_tokens, d_model = x.shape
    n_experts = W_gate.shape[1]
    b2 = b_gate.reshape(1, n_experts)
    grid = n_tokens // BLK

    idx, val, total = pl.pallas_call(
        functools.partial(_router_body, n_experts=n_experts),
        grid=(grid,),
        in_specs=[
            pl.BlockSpec((BLK, d_model), lambda i: (i, 0)),
            pl.BlockSpec((d_model, n_experts), lambda i: (0, 0)),
            pl.BlockSpec((1, n_experts), lambda i: (0, 0)),
        ],
        out_specs=[
            pl.BlockSpec((BLK, TOPK), lambda i: (i, 0)),
            pl.BlockSpec((BLK, TOPK), lambda i: (i, 0)),
            pl.BlockSpec(memory_space=pltpu.SMEM),
        ],
        out_shape=[
            jax.ShapeDtypeStruct((n_tokens, TOPK), jnp.int32),
            jax.ShapeDtypeStruct((n_tokens, TOPK), jnp.float32),
            jax.ShapeDtypeStruct((1,), jnp.float32),
        ],
    )(x, W_gate, b2)

    weights = pl.pallas_call(
        _norm_body,
        in_specs=[
            pl.BlockSpec((n_tokens, TOPK), lambda: (0, 0)),
            pl.BlockSpec(memory_space=pltpu.SMEM),
        ],
        out_specs=pl.BlockSpec((n_tokens, TOPK), lambda: (0, 0)),
        out_shape=jax.ShapeDtypeStruct((n_tokens, TOPK), jnp.float32),
    )(val, total)

    return idx.reshape(-1), weights
